# baseline (speedup 1.0000x reference)
import functools

import jax
import jax.numpy as jnp
from jax import lax
from jax.experimental import pallas as pl
from jax.experimental.pallas import tpu as pltpu
from jax.experimental.pallas import tpu_sc as plsc

_E = 8
_H = 1024
_I = 4096
_N = 4096
_CAP = 640
_SLOTS = _E * _CAP
_SLOTS_PAD = _CAP * (_E + 1)
_EPAD = 128
_TBLK = 512
_NEG = -1e30

_NC, _NS = 2, 16
_NW = _NC * _NS
_TW = _N // _NW
_DC = 32
_CC = 16
_LANES = 16


def _router_body(x_ref, ee_ref, s1_ref, s2_ref, w1_ref, w2_ref, base_ref):
    t = pl.program_id(0)
    x = x_ref[...]
    logits = lax.dot_general(x, ee_ref[...], (((1,), (1,)), ((), ())),
                             preferred_element_type=jnp.float32)
    col = lax.broadcasted_iota(jnp.int32, logits.shape, 1)
    logits = jnp.where(col < _E, logits, _NEG)
    m1 = jnp.max(logits, axis=1, keepdims=True)
    a1 = jnp.min(jnp.where(logits == m1, col, 1 << 30), axis=1, keepdims=True)
    l2 = jnp.where(col == a1, _NEG, logits)
    m2 = jnp.max(l2, axis=1, keepdims=True)
    a2 = jnp.min(jnp.where(l2 == m2, col, 1 << 30), axis=1, keepdims=True)
    r = jnp.exp(m2 - m1)
    w1 = 1.0 / (1.0 + r)
    w2 = r / (1.0 + r)

    onehot = jnp.where((col == a1) | (col == a2), 1.0, 0.0)
    rowi = lax.broadcasted_iota(jnp.int32, (_TBLK, _TBLK), 0)
    coli = lax.broadcasted_iota(jnp.int32, (_TBLK, _TBLK), 1)
    tri = jnp.where(coli <= rowi, 1.0, 0.0)
    csum = lax.dot_general(tri, onehot, (((1,), (0,)), ((), ())),
                           preferred_element_type=jnp.float32)

    @pl.when(t == 0)
    def _():
        base_ref[...] = jnp.zeros_like(base_ref)

    base = base_ref[0:1, :]
    totals = csum + base
    base_ref[...] = jnp.broadcast_to(totals[_TBLK - 1:_TBLK, :], base_ref.shape)

    p1 = jnp.sum(jnp.where(col == a1, totals, 0.0), axis=1, keepdims=True) - 1.0
    p2 = jnp.sum(jnp.where(col == a2, totals, 0.0), axis=1, keepdims=True) - 1.0
    p1i = p1.astype(jnp.int32)
    p2i = p2.astype(jnp.int32)
    row = lax.broadcasted_iota(jnp.int32, (_TBLK, 1), 0)
    sent = _SLOTS + (row % (_SLOTS_PAD - _SLOTS))
    s1 = jnp.where(p1i < _CAP, a1 * _CAP + p1i, sent)
    s2 = jnp.where(p2i < _CAP, a2 * _CAP + p2i, sent)
    w1 = jnp.where(p1i < _CAP, w1, 0.0)
    w2 = jnp.where(p2i < _CAP, w2, 0.0)

    s1_ref[...] = jnp.reshape(s1, (1, 1, _TBLK))
    s2_ref[...] = jnp.reshape(s2, (1, 1, _TBLK))
    w1_ref[...] = jnp.broadcast_to(w1, (_TBLK, _LANES))
    w2_ref[...] = jnp.broadcast_to(w2, (_TBLK, _LANES))


def _router_call(xf, eep):
    nblk = _N // _TBLK
    slot_out = jax.ShapeDtypeStruct((nblk, 1, _TBLK), jnp.int32)
    w_out = jax.ShapeDtypeStruct((_N, _LANES), jnp.float32)
    return pl.pallas_call(
        _router_body,
        grid=(nblk,),
        in_specs=[
            pl.BlockSpec((_TBLK, _H), lambda t: (t, 0)),
            pl.BlockSpec((_EPAD, _H), lambda t: (0, 0)),
        ],
        out_specs=[
            pl.BlockSpec((1, 1, _TBLK), lambda t: (t, 0, 0)),
            pl.BlockSpec((1, 1, _TBLK), lambda t: (t, 0, 0)),
            pl.BlockSpec((_TBLK, _LANES), lambda t: (t, 0)),
            pl.BlockSpec((_TBLK, _LANES), lambda t: (t, 0)),
        ],
        out_shape=[slot_out, slot_out, w_out, w_out],
        scratch_shapes=[pltpu.VMEM((8, _EPAD), jnp.float32)],
    )(xf, eep)


_NDC = _TW // _DC


def _dispatch_body(x_hbm, s1_hbm, s2_hbm, yin_hbm, xb0, xb1, idx,
                   ls0, ls1, ss0, ss1):
    wid = lax.axis_index("s") * _NC + lax.axis_index("c")
    t0 = wid * _TW
    r0 = wid * _NDC
    xbufs = (xb0, xb1)
    lsems = (ls0, ls1)
    ssems = (ss0, ss1)
    lpend = [None, None]
    spend = [(), ()]
    lpend[0] = pltpu.async_copy(x_hbm.at[pl.ds(t0, _DC)], xb0, ls0)
    pltpu.sync_copy(s1_hbm.at[pl.ds(r0, _NDC)], idx.at[pl.ds(0, _NDC)])
    pltpu.sync_copy(s2_hbm.at[pl.ds(r0, _NDC)], idx.at[pl.ds(_NDC, _NDC)])
    for c in range(_NDC):
        p = c & 1
        q = 1 - p
        if c + 1 < _NDC:
            for d in spend[q]:
                d.wait()
            spend[q] = ()
            lpend[q] = pltpu.async_copy(
                x_hbm.at[pl.ds(t0 + (c + 1) * _DC, _DC)], xbufs[q], lsems[q])
        lpend[p].wait()
        spend[p] = (
            pltpu.async_copy(xbufs[p], yin_hbm.at[idx.at[c]], ssems[p]),
            pltpu.async_copy(xbufs[p], yin_hbm.at[idx.at[_NDC + c]], ssems[p]),
        )
    for pair in spend:
        for d in pair:
            d.wait()


def _dispatch_call(xf, s1_2d, s2_2d):
    mesh = plsc.VectorSubcoreMesh(core_axis_name="c", subcore_axis_name="s",
                                  num_cores=_NC, num_subcores=_NS)
    f = pl.kernel(
        _dispatch_body,
        out_type=jax.ShapeDtypeStruct((_SLOTS_PAD, _H), jnp.float32),
        mesh=mesh,
        scratch_types=[
            pltpu.VMEM((_DC, _H), jnp.float32),
            pltpu.VMEM((_DC, _H), jnp.float32),
            pltpu.VMEM((2 * _NDC, _DC), jnp.int32),
            pltpu.SemaphoreType.DMA,
            pltpu.SemaphoreType.DMA,
            pltpu.SemaphoreType.DMA,
            pltpu.SemaphoreType.DMA,
        ],
    )
    return f(xf, s1_2d, s2_2d)


def _ffn_body(a_ref, w1_ref, w2_ref, out_ref):
    i = pl.program_id(1)
    a = a_ref[...]
    a = jnp.where(jnp.abs(a) < 1e6, a, 0.0)
    inter = lax.dot_general(a, w1_ref[0], (((1,), (1,)), ((), ())),
                            preferred_element_type=jnp.float32)
    inter = jnp.maximum(inter, 0.0)
    part = lax.dot_general(inter, w2_ref[0], (((1,), (1,)), ((), ())),
                           preferred_element_type=jnp.float32)

    @pl.when(i == 0)
    def _():
        out_ref[...] = part[None]

    @pl.when(i > 0)
    def _():
        out_ref[...] += part[None]


def _ffn_call(yin, first_linear, second_linear):
    iblk = 1024
    ni = _I // iblk
    return pl.pallas_call(
        _ffn_body,
        grid=(_E, ni),
        in_specs=[
            pl.BlockSpec((_CAP, _H), lambda e, i: (e, 0)),
            pl.BlockSpec((1, iblk, _H), lambda e, i: (e, i, 0)),
            pl.BlockSpec((1, _H, iblk), lambda e, i: (e, 0, i)),
        ],
        out_specs=pl.BlockSpec((1, _CAP, _H), lambda e, i: (e, 0, 0)),
        out_shape=jax.ShapeDtypeStruct((_E, _CAP, _H), jnp.float32),
    )(yin, first_linear, second_linear)


_NCH = _TW // _CC


def _combine_body(y_hbm, s1_hbm, s2_hbm, w1_hbm, w2_hbm, o_hbm,
                  a0, b0, a1, b1, gb1, gb2, wb1, wb2, gs0, gs1, os0, os1):
    wid = lax.axis_index("s") * _NC + lax.axis_index("c")
    t0 = wid * _TW
    pltpu.sync_copy(s1_hbm.at[pl.ds(t0, _TW)], gb1)
    pltpu.sync_copy(s2_hbm.at[pl.ds(t0, _TW)], gb2)
    pltpu.sync_copy(w1_hbm.at[pl.ds(t0 * _LANES, _TW * _LANES)], wb1)
    pltpu.sync_copy(w2_hbm.at[pl.ds(t0 * _LANES, _TW * _LANES)], wb2)
    for j in range(_TW // _LANES):
        sl = pl.ds(j * _LANES, _LANES)
        v1 = gb1[sl]
        v2 = gb2[sl]
        tok = lax.iota(jnp.int32, _LANES) + (t0 + j * _LANES)
        gb1[sl] = jnp.where(v1 < _SLOTS, v1, tok)
        gb2[sl] = jnp.where(v2 < _SLOTS, v2, tok)

    abufs = (a0, a1)
    bbufs = (b0, b1)
    gsems = (gs0, gs1)
    osems = (os0, os1)

    def fire(c, p):
        d1 = pltpu.async_copy(y_hbm.at[gb1.at[pl.ds(c * _CC, _CC)]],
                              abufs[p], gsems[p])
        d2 = pltpu.async_copy(y_hbm.at[gb2.at[pl.ds(c * _CC, _CC)]],
                              bbufs[p], gsems[p])
        return (d1, d2)

    gpend = [None, None]
    opend = [None, None]
    gpend[0] = fire(0, 0)
    for c in range(_NCH):
        p = c & 1
        q = 1 - p
        if c + 1 < _NCH:
            if opend[q] is not None:
                opend[q].wait()
                opend[q] = None
            gpend[q] = fire(c + 1, q)
        for d in gpend[p]:
            d.wait()
        ab = abufs[p]
        bb = bbufs[p]

        def row_body(r, carry):
            wo = (c * _CC + r) * _LANES
            wv1 = wb1[pl.ds(wo, _LANES)]
            wv2 = wb2[pl.ds(wo, _LANES)]
            for j in range(_H // _LANES):
                cs = pl.ds(j * _LANES, _LANES)
                ab[r, cs] = wv1 * ab[r, cs] + wv2 * bb[r, cs]
            return carry

        lax.fori_loop(0, _CC, row_body, 0)
        opend[p] = pltpu.async_copy(ab, o_hbm.at[pl.ds(t0 + c * _CC, _CC)],
                                    osems[p])
    for d in opend:
        if d is not None:
            d.wait()


def _combine_call(yflat, s1, s2, w1, w2):
    mesh = plsc.VectorSubcoreMesh(core_axis_name="c", subcore_axis_name="s",
                                  num_cores=_NC, num_subcores=_NS)
    f = pl.kernel(
        _combine_body,
        out_type=jax.ShapeDtypeStruct((_N, _H), jnp.float32),
        mesh=mesh,
        scratch_types=[
            pltpu.VMEM((_CC, _H), jnp.float32),
            pltpu.VMEM((_CC, _H), jnp.float32),
            pltpu.VMEM((_CC, _H), jnp.float32),
            pltpu.VMEM((_CC, _H), jnp.float32),
            pltpu.VMEM((_TW,), jnp.int32),
            pltpu.VMEM((_TW,), jnp.int32),
            pltpu.VMEM((_TW * _LANES,), jnp.float32),
            pltpu.VMEM((_TW * _LANES,), jnp.float32),
            pltpu.SemaphoreType.DMA,
            pltpu.SemaphoreType.DMA,
            pltpu.SemaphoreType.DMA,
            pltpu.SemaphoreType.DMA,
        ],
    )
    return f(yflat, s1, s2, w1, w2)


def kernel(x, expert_embeddings, first_linear, second_linear):
    b, s, h = x.shape
    xf = x.reshape(b * s, h)
    eep = jnp.zeros((_EPAD, _H), jnp.float32).at[:_E].set(expert_embeddings)
    s1o, s2o, w1o, w2o = _router_call(xf, eep)
    s1 = s1o.reshape(_N)
    s2 = s2o.reshape(_N)
    w1b = w1o.reshape(-1)
    w2b = w2o.reshape(-1)
    yin = _dispatch_call(xf, s1.reshape(_N // _DC, _DC), s2.reshape(_N // _DC, _DC))
    yffn = _ffn_call(yin, first_linear, second_linear)
    yflat = yffn.reshape(_SLOTS, _H)
    out = _combine_call(yflat, s1, s2, w1b, w2b)
    return out.reshape(b, s, h)


# speedup vs baseline: 1.0542x; 1.0542x over previous
import functools

import jax
import jax.numpy as jnp
from jax import lax
from jax.experimental import pallas as pl
from jax.experimental.pallas import tpu as pltpu
from jax.experimental.pallas import tpu_sc as plsc

_E = 8
_H = 1024
_I = 4096
_N = 4096
_CAP = 640
_SLOTS = _E * _CAP
_SLOTS_PAD = _CAP * (_E + 1)
_EPAD = 128
_TBLK = 512
_NEG = -1e30

_NC, _NS = 2, 16
_NW = _NC * _NS
_TW = _N // _NW
_DC = 32
_CC = 16
_LANES = 16


def _router_body(x_ref, ee_ref, s1_ref, s2_ref, w1_ref, w2_ref, base_ref):
    t = pl.program_id(0)
    x = x_ref[...]
    logits = lax.dot_general(x, ee_ref[...], (((1,), (1,)), ((), ())),
                             preferred_element_type=jnp.float32)
    col = lax.broadcasted_iota(jnp.int32, logits.shape, 1)
    logits = jnp.where(col < _E, logits, _NEG)
    m1 = jnp.max(logits, axis=1, keepdims=True)
    a1 = jnp.min(jnp.where(logits == m1, col, 1 << 30), axis=1, keepdims=True)
    l2 = jnp.where(col == a1, _NEG, logits)
    m2 = jnp.max(l2, axis=1, keepdims=True)
    a2 = jnp.min(jnp.where(l2 == m2, col, 1 << 30), axis=1, keepdims=True)
    r = jnp.exp(m2 - m1)
    w1 = 1.0 / (1.0 + r)
    w2 = r / (1.0 + r)

    onehot = jnp.where((col == a1) | (col == a2), 1.0, 0.0)
    rowi = lax.broadcasted_iota(jnp.int32, (_TBLK, _TBLK), 0)
    coli = lax.broadcasted_iota(jnp.int32, (_TBLK, _TBLK), 1)
    tri = jnp.where(coli <= rowi, 1.0, 0.0)
    csum = lax.dot_general(tri, onehot, (((1,), (0,)), ((), ())),
                           preferred_element_type=jnp.float32)

    @pl.when(t == 0)
    def _():
        base_ref[...] = jnp.zeros_like(base_ref)

    base = base_ref[0:1, :]
    totals = csum + base
    base_ref[...] = jnp.broadcast_to(totals[_TBLK - 1:_TBLK, :], base_ref.shape)

    p1 = jnp.sum(jnp.where(col == a1, totals, 0.0), axis=1, keepdims=True) - 1.0
    p2 = jnp.sum(jnp.where(col == a2, totals, 0.0), axis=1, keepdims=True) - 1.0
    p1i = p1.astype(jnp.int32)
    p2i = p2.astype(jnp.int32)
    row = lax.broadcasted_iota(jnp.int32, (_TBLK, 1), 0)
    sent = _SLOTS + (row % (_SLOTS_PAD - _SLOTS))
    s1 = jnp.where(p1i < _CAP, a1 * _CAP + p1i, sent)
    s2 = jnp.where(p2i < _CAP, a2 * _CAP + p2i, sent)
    w1 = jnp.where(p1i < _CAP, w1, 0.0)
    w2 = jnp.where(p2i < _CAP, w2, 0.0)

    s1_ref[...] = jnp.reshape(s1, (1, 1, _TBLK))
    s2_ref[...] = jnp.reshape(s2, (1, 1, _TBLK))
    w1_ref[...] = jnp.broadcast_to(w1, (_TBLK, _LANES))
    w2_ref[...] = jnp.broadcast_to(w2, (_TBLK, _LANES))


def _router_call(xf, eep):
    nblk = _N // _TBLK
    slot_out = jax.ShapeDtypeStruct((nblk, 1, _TBLK), jnp.int32)
    w_out = jax.ShapeDtypeStruct((_N, _LANES), jnp.float32)
    return pl.pallas_call(
        _router_body,
        grid=(nblk,),
        in_specs=[
            pl.BlockSpec((_TBLK, _H), lambda t: (t, 0)),
            pl.BlockSpec((_EPAD, _H), lambda t: (0, 0)),
        ],
        out_specs=[
            pl.BlockSpec((1, 1, _TBLK), lambda t: (t, 0, 0)),
            pl.BlockSpec((1, 1, _TBLK), lambda t: (t, 0, 0)),
            pl.BlockSpec((_TBLK, _LANES), lambda t: (t, 0)),
            pl.BlockSpec((_TBLK, _LANES), lambda t: (t, 0)),
        ],
        out_shape=[slot_out, slot_out, w_out, w_out],
        scratch_shapes=[pltpu.VMEM((8, _EPAD), jnp.float32)],
    )(xf, eep)


_NDC = _TW // _DC


def _dispatch_body(x_hbm, s1_hbm, s2_hbm, yin_hbm, xb0, xb1, idx,
                   ls0, ls1, ss0, ss1):
    wid = lax.axis_index("s") * _NC + lax.axis_index("c")
    t0 = wid * _TW
    r0 = wid * _NDC
    xbufs = (xb0, xb1)
    lsems = (ls0, ls1)
    ssems = (ss0, ss1)
    lpend = [None, None]
    spend = [(), ()]
    lpend[0] = pltpu.async_copy(x_hbm.at[pl.ds(t0, _DC)], xb0, ls0)
    pltpu.sync_copy(s1_hbm.at[pl.ds(r0, _NDC)], idx.at[pl.ds(0, _NDC)])
    pltpu.sync_copy(s2_hbm.at[pl.ds(r0, _NDC)], idx.at[pl.ds(_NDC, _NDC)])
    for c in range(_NDC):
        p = c & 1
        q = 1 - p
        if c + 1 < _NDC:
            for d in spend[q]:
                d.wait()
            spend[q] = ()
            lpend[q] = pltpu.async_copy(
                x_hbm.at[pl.ds(t0 + (c + 1) * _DC, _DC)], xbufs[q], lsems[q])
        lpend[p].wait()
        spend[p] = (
            pltpu.async_copy(xbufs[p], yin_hbm.at[idx.at[c]], ssems[p]),
            pltpu.async_copy(xbufs[p], yin_hbm.at[idx.at[_NDC + c]], ssems[p]),
        )
    for pair in spend:
        for d in pair:
            d.wait()


def _dispatch_call(xf, s1_2d, s2_2d):
    mesh = plsc.VectorSubcoreMesh(core_axis_name="c", subcore_axis_name="s",
                                  num_cores=_NC, num_subcores=_NS)
    f = pl.kernel(
        _dispatch_body,
        out_type=jax.ShapeDtypeStruct((_SLOTS_PAD, _H), jnp.float32),
        mesh=mesh,
        scratch_types=[
            pltpu.VMEM((_DC, _H), jnp.float32),
            pltpu.VMEM((_DC, _H), jnp.float32),
            pltpu.VMEM((2 * _NDC, _DC), jnp.int32),
            pltpu.SemaphoreType.DMA,
            pltpu.SemaphoreType.DMA,
            pltpu.SemaphoreType.DMA,
            pltpu.SemaphoreType.DMA,
        ],
    )
    return f(xf, s1_2d, s2_2d)


def _ffn_body(a_ref, w1_ref, w2_ref, out_ref):
    i = pl.program_id(1)
    a = a_ref[...]
    a = jnp.where(jnp.abs(a) < 1e6, a, 0.0)
    inter = lax.dot_general(a, w1_ref[0], (((1,), (1,)), ((), ())),
                            preferred_element_type=jnp.float32)
    inter = jnp.maximum(inter, 0.0)
    part = lax.dot_general(inter, w2_ref[0], (((1,), (1,)), ((), ())),
                           preferred_element_type=jnp.float32)

    @pl.when(i == 0)
    def _():
        out_ref[...] = part[None]

    @pl.when(i > 0)
    def _():
        out_ref[...] += part[None]


def _ffn_call(yin, first_linear, second_linear):
    iblk = 2048
    ni = _I // iblk
    return pl.pallas_call(
        _ffn_body,
        grid=(_E, ni),
        in_specs=[
            pl.BlockSpec((_CAP, _H), lambda e, i: (e, 0)),
            pl.BlockSpec((1, iblk, _H), lambda e, i: (e, i, 0)),
            pl.BlockSpec((1, _H, iblk), lambda e, i: (e, 0, i)),
        ],
        out_specs=pl.BlockSpec((1, _CAP, _H), lambda e, i: (e, 0, 0)),
        out_shape=jax.ShapeDtypeStruct((_E, _CAP, _H), jnp.float32),
    )(yin, first_linear, second_linear)


_NCH = _TW // _CC


def _combine_body(y_hbm, s1_hbm, s2_hbm, w1_hbm, w2_hbm, o_hbm,
                  a0, b0, a1, b1, gb1, gb2, wb1, wb2, gs0, gs1, os0, os1):
    wid = lax.axis_index("s") * _NC + lax.axis_index("c")
    t0 = wid * _TW
    pltpu.sync_copy(s1_hbm.at[pl.ds(t0, _TW)], gb1)
    pltpu.sync_copy(s2_hbm.at[pl.ds(t0, _TW)], gb2)
    pltpu.sync_copy(w1_hbm.at[pl.ds(t0 * _LANES, _TW * _LANES)], wb1)
    pltpu.sync_copy(w2_hbm.at[pl.ds(t0 * _LANES, _TW * _LANES)], wb2)
    for j in range(_TW // _LANES):
        sl = pl.ds(j * _LANES, _LANES)
        v1 = gb1[sl]
        v2 = gb2[sl]
        tok = lax.iota(jnp.int32, _LANES) + (t0 + j * _LANES)
        gb1[sl] = jnp.where(v1 < _SLOTS, v1, tok)
        gb2[sl] = jnp.where(v2 < _SLOTS, v2, tok)

    abufs = (a0, a1)
    bbufs = (b0, b1)
    gsems = (gs0, gs1)
    osems = (os0, os1)

    def fire(c, p):
        d1 = pltpu.async_copy(y_hbm.at[gb1.at[pl.ds(c * _CC, _CC)]],
                              abufs[p], gsems[p])
        d2 = pltpu.async_copy(y_hbm.at[gb2.at[pl.ds(c * _CC, _CC)]],
                              bbufs[p], gsems[p])
        return (d1, d2)

    gpend = [None, None]
    opend = [None, None]
    gpend[0] = fire(0, 0)
    for c in range(_NCH):
        p = c & 1
        q = 1 - p
        if c + 1 < _NCH:
            if opend[q] is not None:
                opend[q].wait()
                opend[q] = None
            gpend[q] = fire(c + 1, q)
        for d in gpend[p]:
            d.wait()
        ab = abufs[p]
        bb = bbufs[p]

        def row_body(r, carry):
            wo = (c * _CC + r) * _LANES
            wv1 = wb1[pl.ds(wo, _LANES)]
            wv2 = wb2[pl.ds(wo, _LANES)]
            for j in range(_H // _LANES):
                cs = pl.ds(j * _LANES, _LANES)
                ab[r, cs] = wv1 * ab[r, cs] + wv2 * bb[r, cs]
            return carry

        lax.fori_loop(0, _CC, row_body, 0)
        opend[p] = pltpu.async_copy(ab, o_hbm.at[pl.ds(t0 + c * _CC, _CC)],
                                    osems[p])
    for d in opend:
        if d is not None:
            d.wait()


def _combine_call(yflat, s1, s2, w1, w2):
    mesh = plsc.VectorSubcoreMesh(core_axis_name="c", subcore_axis_name="s",
                                  num_cores=_NC, num_subcores=_NS)
    f = pl.kernel(
        _combine_body,
        out_type=jax.ShapeDtypeStruct((_N, _H), jnp.float32),
        mesh=mesh,
        scratch_types=[
            pltpu.VMEM((_CC, _H), jnp.float32),
            pltpu.VMEM((_CC, _H), jnp.float32),
            pltpu.VMEM((_CC, _H), jnp.float32),
            pltpu.VMEM((_CC, _H), jnp.float32),
            pltpu.VMEM((_TW,), jnp.int32),
            pltpu.VMEM((_TW,), jnp.int32),
            pltpu.VMEM((_TW * _LANES,), jnp.float32),
            pltpu.VMEM((_TW * _LANES,), jnp.float32),
            pltpu.SemaphoreType.DMA,
            pltpu.SemaphoreType.DMA,
            pltpu.SemaphoreType.DMA,
            pltpu.SemaphoreType.DMA,
        ],
    )
    return f(yflat, s1, s2, w1, w2)


def kernel(x, expert_embeddings, first_linear, second_linear):
    b, s, h = x.shape
    xf = x.reshape(b * s, h)
    eep = jnp.zeros((_EPAD, _H), jnp.float32).at[:_E].set(expert_embeddings)
    s1o, s2o, w1o, w2o = _router_call(xf, eep)
    s1 = s1o.reshape(_N)
    s2 = s2o.reshape(_N)
    w1b = w1o.reshape(-1)
    w2b = w2o.reshape(-1)
    yin = _dispatch_call(xf, s1.reshape(_N // _DC, _DC), s2.reshape(_N // _DC, _DC))
    yffn = _ffn_call(yin, first_linear, second_linear)
    yflat = yffn.reshape(_SLOTS, _H)
    out = _combine_call(yflat, s1, s2, w1b, w2b)
    return out.reshape(b, s, h)
